# baseline (speedup 1.0000x reference)
import functools

import jax
import jax.numpy as jnp
from jax import lax
from jax.experimental import pallas as pl
from jax.experimental.pallas import tpu as pltpu
from jax.experimental.pallas import tpu_sc as plsc

_NUM_PT = 32


def _transpose_table(rules):
    num_pt, v = rules.shape
    c = 4096

    def body(r_ref, o_ref):
        o_ref[...] = r_ref[...].T

    return pl.pallas_call(
        body,
        grid=(pl.cdiv(v, c),),
        in_specs=[pl.BlockSpec((num_pt, c), lambda i: (0, i))],
        out_specs=pl.BlockSpec((c, num_pt), lambda i: (i, 0)),
        out_shape=jax.ShapeDtypeStruct((v, num_pt), jnp.float32),
    )(rules)


def _sc_gather_transpose(table_t, sentences):
    b, l = sentences.shape
    idx_flat = sentences.reshape(-1)
    n = b * l
    info = plsc.get_sparse_core_info()
    nc, ns = info.num_cores, info.num_subcores
    nw = nc * ns
    s_chunk = 4
    t_chunk = s_chunk * l
    n_chunks = b // (nw * s_chunk)
    n_groups = t_chunk // 16
    mesh = plsc.VectorSubcoreMesh(core_axis_name="c", subcore_axis_name="s")

    @functools.partial(
        pl.kernel,
        mesh=mesh,
        out_type=jax.ShapeDtypeStruct((b, _NUM_PT, l), jnp.float32),
        scratch_types=[
            pltpu.VMEM((t_chunk,), jnp.int32),
            pltpu.VMEM((t_chunk, _NUM_PT), jnp.float32),
            pltpu.VMEM((s_chunk, _NUM_PT, l), jnp.float32),
            pltpu.SemaphoreType.DMA,
        ],
        compiler_params=pltpu.CompilerParams(
            use_tc_tiling_on_sc=False, needs_layout_passes=False
        ),
    )
    def k(table_hbm, idx_hbm, out_hbm, idx_v, rows_v, outb_v, sem):
        wid = lax.axis_index("s") * nc + lax.axis_index("c")

        def chunk_body(ci, carry):
            sent0 = (wid * n_chunks + ci) * s_chunk
            tok0 = sent0 * l
            pltpu.sync_copy(idx_hbm.at[pl.ds(tok0, t_chunk)], idx_v)
            pltpu.async_copy(table_hbm.at[idx_v], rows_v, sem).wait()

            def group_body(g, carry2):
                i = g * 16 + lax.iota(jnp.int32, 16)
                b_loc = (i * 41) >> 13
                l_loc = i - b_loc * 200
                for pt in range(_NUM_PT):
                    pt_v = jnp.full((16,), pt, jnp.int32)
                    vals = plsc.load_gather(rows_v, [i, pt_v])
                    plsc.store_scatter(outb_v, [b_loc, pt_v, l_loc], vals)
                return carry2

            lax.fori_loop(0, n_groups, group_body, 0)
            pltpu.sync_copy(outb_v, out_hbm.at[pl.ds(sent0, s_chunk)])
            return carry

        lax.fori_loop(0, n_chunks, chunk_body, 0)

    return k(table_t, idx_flat)


def kernel(sentences, rules):
    table_t = _transpose_table(rules)
    return _sc_gather_transpose(table_t, sentences)


# speedup vs baseline: 1.5689x; 1.5689x over previous
import functools

import jax
import jax.numpy as jnp
from jax import lax
from jax.experimental import pallas as pl
from jax.experimental.pallas import tpu as pltpu
from jax.experimental.pallas import tpu_sc as plsc

_NUM_PT = 32


def _transpose_table(rules):
    num_pt, v = rules.shape
    c = 4096
    nblk = pl.cdiv(v, c)

    def body(r_ref, o_ref):
        for j in range(4):
            o_ref[:, 32 * j:32 * (j + 1)] = r_ref[:, 1024 * j:1024 * (j + 1)].T

    return pl.pallas_call(
        body,
        grid=(nblk,),
        in_specs=[pl.BlockSpec((num_pt, c), lambda i: (0, i))],
        out_specs=pl.BlockSpec((c // 4, 4 * num_pt), lambda i: (i, 0)),
        out_shape=jax.ShapeDtypeStruct((nblk * c // 4, 4 * num_pt), jnp.float32),
    )(rules)


def _sc_gather_transpose(table_t, sentences):
    b, l = sentences.shape
    idx_flat = sentences.reshape(-1)
    info = plsc.get_sparse_core_info()
    nc, ns = info.num_cores, info.num_subcores
    nw = nc * ns
    s_chunk = 4
    t_chunk = s_chunk * l
    n_chunks = b // (nw * s_chunk)
    n_groups = t_chunk // 16
    mesh = plsc.VectorSubcoreMesh(core_axis_name="c", subcore_axis_name="s")

    @functools.partial(
        pl.kernel,
        mesh=mesh,
        out_type=jax.ShapeDtypeStruct((b * _NUM_PT * l,), jnp.float32),
        scratch_types=[
            pltpu.VMEM((t_chunk,), jnp.int32),
            pltpu.VMEM((t_chunk, _NUM_PT), jnp.float32),
            pltpu.VMEM((s_chunk * _NUM_PT * l,), jnp.float32),
            pltpu.SemaphoreType.DMA,
        ],
        compiler_params=pltpu.CompilerParams(
            use_tc_tiling_on_sc=False,
            needs_layout_passes=False,
            disable_bounds_checks=True,
        ),
    )
    def k(table_hbm, idx_hbm, out_hbm, idx_v, rows_v, outb_v, sem):
        wid = lax.axis_index("s") * nc + lax.axis_index("c")
        iota = lax.iota(jnp.int32, 16)

        def chunk_body(ci, carry):
            sent0 = (wid * n_chunks + ci) * s_chunk
            tok0 = sent0 * l
            pltpu.sync_copy(idx_hbm.at[pl.ds(tok0, t_chunk)], idx_v)

            @plsc.parallel_loop(0, n_groups)
            def remap_body(g):
                t = idx_v[pl.ds(g * 16, 16)]
                t2 = ((t & jnp.int32(-4096)) + ((t & 1023) << 2)
                      + ((t >> 10) & 3))
                idx_v[pl.ds(g * 16, 16)] = t2

            pltpu.async_copy(table_hbm.at[idx_v], rows_v, sem).wait()

            @plsc.parallel_loop(0, n_groups, unroll=2)
            def group_body(g):
                i = g * 16 + iota
                b_loc = (i * 41) >> 13
                dst0 = i + b_loc * (_NUM_PT * l - l)
                for pt in range(_NUM_PT):
                    pt_v = jnp.full((16,), pt, jnp.int32)
                    vals = plsc.load_gather(rows_v, [i, pt_v])
                    plsc.store_scatter(outb_v, [dst0 + pt * l], vals)

            pltpu.sync_copy(
                outb_v, out_hbm.at[pl.ds(tok0 * _NUM_PT, t_chunk * _NUM_PT)]
            )
            return carry

        lax.fori_loop(0, n_chunks, chunk_body, 0)

    return k(table_t, idx_flat).reshape(b, _NUM_PT, l)


def kernel(sentences, rules):
    b, l = sentences.shape
    table_t = _transpose_table(rules).reshape(-1, _NUM_PT)
    return _sc_gather_transpose(table_t, sentences)


# speedup vs baseline: 1.6864x; 1.0749x over previous
import functools

import jax
import jax.numpy as jnp
from jax import lax
from jax.experimental import pallas as pl
from jax.experimental.pallas import tpu as pltpu
from jax.experimental.pallas import tpu_sc as plsc

_NUM_PT = 32


def _transpose_table(rules):
    num_pt, v = rules.shape
    c = 4096
    nblk = pl.cdiv(v, c)

    def body(r_ref, o_ref):
        for j in range(4):
            o_ref[:, 32 * j:32 * (j + 1)] = r_ref[:, 1024 * j:1024 * (j + 1)].T

    return pl.pallas_call(
        body,
        grid=(nblk,),
        in_specs=[pl.BlockSpec((num_pt, c), lambda i: (0, i))],
        out_specs=pl.BlockSpec((c // 4, 4 * num_pt), lambda i: (i, 0)),
        out_shape=jax.ShapeDtypeStruct((nblk * c // 4, 4 * num_pt), jnp.float32),
    )(rules)


def _sc_gather_transpose(table_t, sentences):
    b, l = sentences.shape
    idx_flat = sentences.reshape(-1)
    info = plsc.get_sparse_core_info()
    nc, ns = info.num_cores, info.num_subcores
    nw = nc * ns
    s_chunk = 4
    t_chunk = s_chunk * l
    n_chunks = b // (nw * s_chunk)
    n_groups = t_chunk // 16
    mesh = plsc.VectorSubcoreMesh(core_axis_name="c", subcore_axis_name="s")

    per_w = n_chunks * t_chunk

    @functools.partial(
        pl.kernel,
        mesh=mesh,
        out_type=jax.ShapeDtypeStruct((b * _NUM_PT * l,), jnp.float32),
        scratch_types=[
            pltpu.VMEM((per_w,), jnp.int32),
            pltpu.VMEM((t_chunk, _NUM_PT), jnp.float32),
            pltpu.VMEM((t_chunk, _NUM_PT), jnp.float32),
            pltpu.VMEM((s_chunk * _NUM_PT * l,), jnp.float32),
            pltpu.SemaphoreType.DMA,
            pltpu.SemaphoreType.DMA,
        ],
        compiler_params=pltpu.CompilerParams(
            use_tc_tiling_on_sc=False,
            needs_layout_passes=False,
            disable_bounds_checks=True,
        ),
    )
    def k(table_hbm, idx_hbm, out_hbm, idx_v, rows_a, rows_b, outb_v,
          sem_a, sem_b):
        wid = lax.axis_index("s") * nc + lax.axis_index("c")
        iota = lax.iota(jnp.int32, 16)

        pltpu.sync_copy(idx_hbm.at[pl.ds(wid * per_w, per_w)], idx_v)

        @plsc.parallel_loop(0, per_w // 16)
        def remap_body(g):
            t = idx_v[pl.ds(g * 16, 16)]
            idx_v[pl.ds(g * 16, 16)] = (
                (t & jnp.int32(-4096)) + ((t & 1023) << 2) + ((t >> 10) & 3)
            )

        def gather(ci, rows, sem):
            return pltpu.make_async_copy(
                table_hbm.at[idx_v.at[pl.ds(ci * t_chunk, t_chunk)]], rows, sem
            )

        def transpose_out(ci, rows):
            @plsc.parallel_loop(0, n_groups, unroll=2)
            def group_body(g):
                i = g * 16 + iota
                b_loc = (i * 41) >> 13
                dst0 = i + b_loc * (_NUM_PT * l - l)
                for pt in range(_NUM_PT):
                    pt_v = jnp.full((16,), pt, jnp.int32)
                    vals = plsc.load_gather(rows, [i, pt_v])
                    plsc.store_scatter(outb_v, [dst0 + pt * l], vals)

            tok0 = (wid * n_chunks + ci) * t_chunk
            pltpu.sync_copy(
                outb_v, out_hbm.at[pl.ds(tok0 * _NUM_PT, t_chunk * _NUM_PT)]
            )

        gather(0, rows_a, sem_a).start()

        def super_body(s, carry):
            c0 = 2 * s
            gather(c0 + 1, rows_b, sem_b).start()
            gather(c0, rows_a, sem_a).wait()
            transpose_out(c0, rows_a)

            @pl.when(c0 + 2 < n_chunks)
            def _():
                gather(c0 + 2, rows_a, sem_a).start()

            gather(c0 + 1, rows_b, sem_b).wait()
            transpose_out(c0 + 1, rows_b)
            return carry

        lax.fori_loop(0, n_chunks // 2, super_body, 0)

    return k(table_t, idx_flat).reshape(b, _NUM_PT, l)


def kernel(sentences, rules):
    b, l = sentences.shape
    table_t = _transpose_table(rules).reshape(-1, _NUM_PT)
    return _sc_gather_transpose(table_t, sentences)


# speedup vs baseline: 1.8177x; 1.0779x over previous
import functools

import jax
import jax.numpy as jnp
from jax import lax
from jax.experimental import pallas as pl
from jax.experimental.pallas import tpu as pltpu
from jax.experimental.pallas import tpu_sc as plsc

_NUM_PT = 32
_TBLK = 8192


def _transpose_table(rules):
    num_pt, v = rules.shape
    c = _TBLK
    q = c // 4
    nblk = pl.cdiv(v, c)

    def body(r_ref, o_ref):
        for j in range(4):
            o_ref[:, 32 * j:32 * (j + 1)] = r_ref[:, q * j:q * (j + 1)].T

    return pl.pallas_call(
        body,
        grid=(nblk,),
        in_specs=[pl.BlockSpec((num_pt, c), lambda i: (0, i))],
        out_specs=pl.BlockSpec((c // 4, 4 * num_pt), lambda i: (i, 0)),
        out_shape=jax.ShapeDtypeStruct((nblk * c // 4, 4 * num_pt), jnp.float32),
    )(rules)


def _sc_gather_transpose(table_t, sentences):
    b, l = sentences.shape
    idx_flat = sentences.reshape(-1)
    info = plsc.get_sparse_core_info()
    nc, ns = info.num_cores, info.num_subcores
    nw = nc * ns
    s_chunk = 4
    t_chunk = s_chunk * l
    n_chunks = b // (nw * s_chunk)
    n_groups = t_chunk // 16
    mesh = plsc.VectorSubcoreMesh(core_axis_name="c", subcore_axis_name="s")

    per_w = n_chunks * t_chunk

    @functools.partial(
        pl.kernel,
        mesh=mesh,
        out_type=jax.ShapeDtypeStruct((b * _NUM_PT * l,), jnp.float32),
        scratch_types=[
            pltpu.VMEM((per_w,), jnp.int32),
            pltpu.VMEM((t_chunk, _NUM_PT), jnp.float32),
            pltpu.VMEM((t_chunk, _NUM_PT), jnp.float32),
            pltpu.VMEM((s_chunk * _NUM_PT * l,), jnp.float32),
            pltpu.VMEM((s_chunk * _NUM_PT * l,), jnp.float32),
            pltpu.SemaphoreType.DMA,
            pltpu.SemaphoreType.DMA,
            pltpu.SemaphoreType.DMA,
            pltpu.SemaphoreType.DMA,
        ],
        compiler_params=pltpu.CompilerParams(
            use_tc_tiling_on_sc=False,
            needs_layout_passes=False,
            disable_bounds_checks=True,
        ),
    )
    def k(table_hbm, idx_hbm, out_hbm, idx_v, rows_a, rows_b, outb_a, outb_b,
          sem_a, sem_b, sem_oa, sem_ob):
        wid = lax.axis_index("s") * nc + lax.axis_index("c")
        iota = lax.iota(jnp.int32, 16)

        pltpu.sync_copy(idx_hbm.at[pl.ds(wid * per_w, per_w)], idx_v)

        qm = _TBLK // 4 - 1
        js = qm.bit_length()

        @plsc.parallel_loop(0, per_w // 16)
        def remap_body(g):
            t = idx_v[pl.ds(g * 16, 16)]
            idx_v[pl.ds(g * 16, 16)] = (
                (t & jnp.int32(-_TBLK)) + ((t & qm) << 2) + ((t >> js) & 3)
            )

        def gather(ci, rows, sem):
            return pltpu.make_async_copy(
                table_hbm.at[idx_v.at[pl.ds(ci * t_chunk, t_chunk)]], rows, sem
            )

        def out_copy(ci, outb, sem_o):
            tok0 = (wid * n_chunks + ci) * t_chunk
            return pltpu.make_async_copy(
                outb, out_hbm.at[pl.ds(tok0 * _NUM_PT, t_chunk * _NUM_PT)],
                sem_o,
            )

        def transpose(rows, outb):
            @plsc.parallel_loop(0, n_groups, unroll=2)
            def group_body(g):
                i = g * 16 + iota
                b_loc = (i * 41) >> 13
                dst0 = i + b_loc * (_NUM_PT * l - l)
                for pt in range(_NUM_PT):
                    pt_v = jnp.full((16,), pt, jnp.int32)
                    vals = plsc.load_gather(rows, [i, pt_v])
                    plsc.store_scatter(outb, [dst0 + pt * l], vals)

        gather(0, rows_a, sem_a).start()

        def super_body(s, carry):
            c0 = 2 * s
            gather(c0 + 1, rows_b, sem_b).start()
            gather(c0, rows_a, sem_a).wait()

            @pl.when(s > 0)
            def _():
                out_copy(c0 - 2, outb_a, sem_oa).wait()

            transpose(rows_a, outb_a)
            out_copy(c0, outb_a, sem_oa).start()

            @pl.when(c0 + 2 < n_chunks)
            def _():
                gather(c0 + 2, rows_a, sem_a).start()

            gather(c0 + 1, rows_b, sem_b).wait()

            @pl.when(s > 0)
            def _():
                out_copy(c0 - 1, outb_b, sem_ob).wait()

            transpose(rows_b, outb_b)
            out_copy(c0 + 1, outb_b, sem_ob).start()
            return carry

        lax.fori_loop(0, n_chunks // 2, super_body, 0)
        out_copy(n_chunks - 2, outb_a, sem_oa).wait()
        out_copy(n_chunks - 1, outb_b, sem_ob).wait()

    return k(table_t, idx_flat).reshape(b, _NUM_PT, l)


def kernel(sentences, rules):
    b, l = sentences.shape
    table_t = _transpose_table(rules).reshape(-1, _NUM_PT)
    return _sc_gather_transpose(table_t, sentences)
